# baseline (speedup 1.0000x reference)
import functools

import jax
import jax.numpy as jnp
from jax.experimental import pallas as pl
from jax.experimental.pallas import tpu as pltpu


def _support_body(h_ref, w_ref, o_ref):
    o_ref[...] = jax.lax.dot(
        h_ref[...], w_ref[...], preferred_element_type=jnp.float32
    ).astype(o_ref.dtype)


def _support_matmul(h, w, bm):
    m, k = h.shape
    n = w.shape[1]
    grid = (pl.cdiv(m, bm),)
    return pl.pallas_call(
        _support_body,
        grid=grid,
        in_specs=[
            pl.BlockSpec((bm, k), lambda i: (i, 0)),
            pl.BlockSpec((k, n), lambda i: (0, 0)),
        ],
        out_specs=pl.BlockSpec((bm, n), lambda i: (i, 0)),
        out_shape=jax.ShapeDtypeStruct((m, n), jnp.bfloat16),
        compiler_params=pltpu.CompilerParams(
            dimension_semantics=("parallel",),
        ),
    )(h, w)


def _adj_body(a_ref, s_ref, b_ref, o_ref, acc_ref, *, nk, kdim, bk, relu):
    k = pl.program_id(1)

    @pl.when(k == 0)
    def _init():
        acc_ref[...] = jnp.zeros_like(acc_ref)

    @pl.when(k < nk - 1)
    def _full_step():
        acc_ref[...] += jax.lax.dot(
            a_ref[...], s_ref[...], preferred_element_type=jnp.float32
        )

    @pl.when(k == nk - 1)
    def _last_step():
        valid = kdim - (nk - 1) * bk
        a = a_ref[...]
        col = jax.lax.broadcasted_iota(jnp.int32, a.shape, 1)
        a = jnp.where(col < valid, a, jnp.zeros_like(a))
        s = s_ref[...]
        row = jax.lax.broadcasted_iota(jnp.int32, s.shape, 0)
        s = jnp.where(row < valid, s, jnp.zeros_like(s))
        acc = acc_ref[...] + jax.lax.dot(
            a, s, preferred_element_type=jnp.float32
        )
        r = acc + b_ref[...]
        if relu:
            r = jnp.maximum(r, 0.0)
        o_ref[...] = r.astype(o_ref.dtype)


def _adj_matmul(adj, support, bias2d, relu, out_dtype, bm, bk):
    m, kdim = adj.shape
    n = support.shape[1]
    nk = pl.cdiv(kdim, bk)
    grid = (pl.cdiv(m, bm), nk)
    return pl.pallas_call(
        functools.partial(_adj_body, nk=nk, kdim=kdim, bk=bk, relu=relu),
        grid=grid,
        in_specs=[
            pl.BlockSpec((bm, bk), lambda i, j: (i, j)),
            pl.BlockSpec((bk, n), lambda i, j: (j, 0)),
            pl.BlockSpec((1, n), lambda i, j: (0, 0)),
        ],
        out_specs=pl.BlockSpec((bm, n), lambda i, j: (i, 0)),
        out_shape=jax.ShapeDtypeStruct((m, n), out_dtype),
        scratch_shapes=[pltpu.VMEM((bm, n), jnp.float32)],
        compiler_params=pltpu.CompilerParams(
            dimension_semantics=("parallel", "arbitrary"),
        ),
    )(adj, support, bias2d)


def _gcn_forward(x, adj, layer_params, bm, bk):
    adj_bf = adj.astype(jnp.bfloat16)
    h = x.astype(jnp.bfloat16)
    n_layers = len(layer_params)
    for i, (w, b) in enumerate(layer_params):
        relu = i < n_layers - 1
        support = _support_matmul(h, w.astype(jnp.bfloat16), bm)
        out_dtype = jnp.bfloat16 if relu else jnp.float32
        h = _adj_matmul(adj_bf, support, b.reshape(1, -1), relu, out_dtype, bm, bk)
    return h


def kernel(x, adj, W1, b1, W2, b2, W3, b3, W4, b4):
    return _gcn_forward(
        x, adj, [(W1, b1), (W2, b2), (W3, b3), (W4, b4)], bm=1024, bk=1024
    )


# speedup vs baseline: 1.3813x; 1.3813x over previous
import functools

import jax
import jax.numpy as jnp
from jax.experimental import pallas as pl
from jax.experimental.pallas import tpu as pltpu

_BM = 1024
_BK = 2048
_QSCALE = 255.0


def _support_body(h_ref, w_ref, o_ref, *, m_valid, bm):
    i = pl.program_id(0)
    h = h_ref[...].astype(jnp.bfloat16)
    w = w_ref[...].astype(jnp.bfloat16)
    o = jax.lax.dot(h, w, preferred_element_type=jnp.float32)
    row = jax.lax.broadcasted_iota(jnp.int32, o.shape, 0) + i * bm
    o = jnp.where(row < m_valid, o, 0.0)
    o_ref[...] = o.astype(o_ref.dtype)


def _support_matmul(h, w, m_pad, bm):
    m, k = h.shape
    n = w.shape[1]
    grid = (m_pad // bm,)
    return pl.pallas_call(
        functools.partial(_support_body, m_valid=m, bm=bm),
        grid=grid,
        in_specs=[
            pl.BlockSpec((bm, k), lambda i: (i, 0)),
            pl.BlockSpec((k, n), lambda i: (0, 0)),
        ],
        out_specs=pl.BlockSpec((bm, n), lambda i: (i, 0)),
        out_shape=jax.ShapeDtypeStruct((m_pad, n), jnp.bfloat16),
        compiler_params=pltpu.CompilerParams(
            dimension_semantics=("parallel",),
        ),
    )(h, w)


def _adj_q_body(a_ref, s_ref, b_ref, q_ref, o_ref, acc_ref, *,
                nk, bk, k_valid, relu):
    j = pl.program_id(1)
    a = a_ref[...]
    qf = jnp.floor(a * _QSCALE + 0.5)
    col = jax.lax.broadcasted_iota(jnp.int32, qf.shape, 1) + j * bk
    qf = jnp.where(col < k_valid, qf, 0.0)
    q_ref[...] = qf.astype(jnp.uint8)

    @pl.when(j == 0)
    def _init():
        acc_ref[...] = jnp.zeros_like(acc_ref)

    s = s_ref[pl.ds(j * bk, bk), :]
    acc_ref[...] += jax.lax.dot(
        qf.astype(jnp.bfloat16), s, preferred_element_type=jnp.float32
    )

    @pl.when(j == nk - 1)
    def _done():
        r = acc_ref[...] * (1.0 / _QSCALE) + b_ref[...]
        if relu:
            r = jnp.maximum(r, 0.0)
        o_ref[...] = r.astype(o_ref.dtype)


def _adj_matmul_quantize(adj, support, bias2d, relu, out_dtype, bm, bk):
    m, kdim = adj.shape
    k_pad, n = support.shape
    nk = k_pad // bk
    grid = (pl.cdiv(m, bm), nk)
    return pl.pallas_call(
        functools.partial(_adj_q_body, nk=nk, bk=bk, k_valid=kdim, relu=relu),
        grid=grid,
        in_specs=[
            pl.BlockSpec((bm, bk), lambda i, j: (i, j)),
            pl.BlockSpec((k_pad, n), lambda i, j: (0, 0)),
            pl.BlockSpec((1, n), lambda i, j: (0, 0)),
        ],
        out_specs=[
            pl.BlockSpec((bm, bk), lambda i, j: (i, j)),
            pl.BlockSpec((bm, n), lambda i, j: (i, 0)),
        ],
        out_shape=[
            jax.ShapeDtypeStruct((m, k_pad), jnp.uint8),
            jax.ShapeDtypeStruct((m, n), out_dtype),
        ],
        scratch_shapes=[pltpu.VMEM((bm, n), jnp.float32)],
        compiler_params=pltpu.CompilerParams(
            dimension_semantics=("parallel", "arbitrary"),
        ),
    )(adj, support, bias2d)


def _adj_u8_body(a_ref, s_ref, b_ref, o_ref, acc_ref, *, nk, bk, relu):
    j = pl.program_id(1)

    @pl.when(j == 0)
    def _init():
        acc_ref[...] = jnp.zeros_like(acc_ref)

    a = a_ref[...].astype(jnp.bfloat16)
    s = s_ref[pl.ds(j * bk, bk), :]
    acc_ref[...] += jax.lax.dot(a, s, preferred_element_type=jnp.float32)

    @pl.when(j == nk - 1)
    def _done():
        r = acc_ref[...] * (1.0 / _QSCALE) + b_ref[...]
        if relu:
            r = jnp.maximum(r, 0.0)
        o_ref[...] = r.astype(o_ref.dtype)


def _adj_matmul_u8(adj_q, support, bias2d, relu, out_dtype, bm, bk):
    m, k_pad = adj_q.shape
    n = support.shape[1]
    nk = k_pad // bk
    grid = (pl.cdiv(m, bm), nk)
    return pl.pallas_call(
        functools.partial(_adj_u8_body, nk=nk, bk=bk, relu=relu),
        grid=grid,
        in_specs=[
            pl.BlockSpec((bm, bk), lambda i, j: (i, j)),
            pl.BlockSpec((k_pad, n), lambda i, j: (0, 0)),
            pl.BlockSpec((1, n), lambda i, j: (0, 0)),
        ],
        out_specs=pl.BlockSpec((bm, n), lambda i, j: (i, 0)),
        out_shape=jax.ShapeDtypeStruct((m, n), out_dtype),
        scratch_shapes=[pltpu.VMEM((bm, n), jnp.float32)],
        compiler_params=pltpu.CompilerParams(
            dimension_semantics=("parallel", "arbitrary"),
        ),
    )(adj_q, support, bias2d)


def _gcn_forward(x, adj, layer_params, bm, bk):
    n_nodes = adj.shape[0]
    k_pad = pl.cdiv(n_nodes, bk) * bk
    n_layers = len(layer_params)
    h = x
    adj_q = None
    for i, (w, b) in enumerate(layer_params):
        relu = i < n_layers - 1
        support = _support_matmul(h, w, k_pad, bm)
        out_dtype = jnp.bfloat16 if relu else jnp.float32
        b2 = b.reshape(1, -1)
        if i == 0:
            adj_q, h = _adj_matmul_quantize(
                adj, support, b2, relu, out_dtype, bm, bk)
        else:
            h = _adj_matmul_u8(adj_q, support, b2, relu, out_dtype, bm, bk)
    return h


def kernel(x, adj, W1, b1, W2, b2, W3, b3, W4, b4):
    return _gcn_forward(
        x, adj, [(W1, b1), (W2, b2), (W3, b3), (W4, b4)], bm=_BM, bk=_BK
    )


# speedup vs baseline: 1.6532x; 1.1968x over previous
import functools

import jax
import jax.numpy as jnp
from jax.experimental import pallas as pl
from jax.experimental.pallas import tpu as pltpu

_BM = 1024
_BK = 2048
_F8_TARGET = 240.0
_F8 = jnp.float8_e4m3fn


def _support_body(h_ref, w_ref, o_ref, mx_ref, cs_ref, *, m_valid, bm):
    i = pl.program_id(0)
    h = h_ref[...].astype(jnp.bfloat16)
    w = w_ref[...].astype(jnp.bfloat16)
    o = jax.lax.dot(h, w, preferred_element_type=jnp.float32)
    row = jax.lax.broadcasted_iota(jnp.int32, o.shape, 0) + i * bm
    o = jnp.where(row < m_valid, o, 0.0)
    o_ref[...] = o.astype(o_ref.dtype)
    mx_ref[...] = jnp.full(mx_ref.shape, jnp.max(jnp.abs(o)), jnp.float32)
    cs_ref[...] = jnp.sum(o, axis=0, keepdims=True)[None]


def _support_matmul(h, w, m_pad, bm):
    m, k = h.shape
    n = w.shape[1]
    ng = m_pad // bm
    return pl.pallas_call(
        functools.partial(_support_body, m_valid=m, bm=bm),
        grid=(ng,),
        in_specs=[
            pl.BlockSpec((bm, k), lambda i: (i, 0)),
            pl.BlockSpec((k, n), lambda i: (0, 0)),
        ],
        out_specs=[
            pl.BlockSpec((bm, n), lambda i: (i, 0)),
            pl.BlockSpec((1, 1, 128), lambda i: (i, 0, 0)),
            pl.BlockSpec((1, 1, n), lambda i: (i, 0, 0)),
        ],
        out_shape=[
            jax.ShapeDtypeStruct((m_pad, n), jnp.bfloat16),
            jax.ShapeDtypeStruct((ng, 1, 128), jnp.float32),
            jax.ShapeDtypeStruct((ng, 1, n), jnp.float32),
        ],
        compiler_params=pltpu.CompilerParams(
            dimension_semantics=("parallel",),
        ),
    )(h, w)


def _quant_body(s_ref, inv_ref, q_ref, cs_ref):
    q = (s_ref[...].astype(jnp.float32) * inv_ref[...]).astype(_F8)
    q_ref[...] = q
    cs_ref[...] = jnp.sum(q.astype(jnp.float32), axis=0, keepdims=True)[None]


def _quant_support(s, inv_row, bm):
    m_pad, n = s.shape
    ng = m_pad // bm
    return pl.pallas_call(
        _quant_body,
        grid=(ng,),
        in_specs=[
            pl.BlockSpec((bm, n), lambda i: (i, 0)),
            pl.BlockSpec((1, n), lambda i: (0, 0)),
        ],
        out_specs=[
            pl.BlockSpec((bm, n), lambda i: (i, 0)),
            pl.BlockSpec((1, 1, n), lambda i: (i, 0, 0)),
        ],
        out_shape=[
            jax.ShapeDtypeStruct((m_pad, n), _F8),
            jax.ShapeDtypeStruct((ng, 1, n), jnp.float32),
        ],
        compiler_params=pltpu.CompilerParams(
            dimension_semantics=("parallel",),
        ),
    )(s, inv_row)


def _adj_q_body(a_ref, s_ref, sv_ref, cr_ref, b_ref, q_ref, rs_ref, o_ref,
                acc_ref, rsacc_ref, *, nk, bk, k_valid, relu):
    j = pl.program_id(1)
    a = a_ref[...]
    col = jax.lax.broadcasted_iota(jnp.int32, a.shape, 1) + j * bk
    a = jnp.where(col < k_valid, a, 0.0)
    qa = a.astype(_F8)
    q_ref[...] = qa

    @pl.when(j == 0)
    def _init():
        acc_ref[...] = jnp.zeros_like(acc_ref)
        rsacc_ref[...] = jnp.zeros_like(rsacc_ref)

    qa32 = qa.astype(jnp.float32)
    rsacc_ref[...] += jnp.broadcast_to(
        jnp.sum(qa32, axis=1, keepdims=True), rsacc_ref.shape)
    s = s_ref[pl.ds(j * bk, bk), :]
    acc_ref[...] += jax.lax.dot(qa, s, preferred_element_type=jnp.float32)

    @pl.when(j == nk - 1)
    def _done():
        rs_ref[...] = rsacc_ref[...]
        r = (acc_ref[...] * sv_ref[...]
             + rsacc_ref[:, 0:1] * cr_ref[...] + b_ref[...])
        if relu:
            r = jnp.maximum(r, 0.0)
        o_ref[...] = r.astype(o_ref.dtype)


def _adj_matmul_quantize(adj, s_q, scale_row, corr_row, bias2d, relu,
                         out_dtype, bm, bk):
    m, kdim = adj.shape
    k_pad, n = s_q.shape
    nk = k_pad // bk
    grid = (pl.cdiv(m, bm), nk)
    return pl.pallas_call(
        functools.partial(_adj_q_body, nk=nk, bk=bk, k_valid=kdim, relu=relu),
        grid=grid,
        in_specs=[
            pl.BlockSpec((bm, bk), lambda i, j: (i, j)),
            pl.BlockSpec((k_pad, n), lambda i, j: (0, 0)),
            pl.BlockSpec((1, n), lambda i, j: (0, 0)),
            pl.BlockSpec((1, n), lambda i, j: (0, 0)),
            pl.BlockSpec((1, n), lambda i, j: (0, 0)),
        ],
        out_specs=[
            pl.BlockSpec((bm, bk), lambda i, j: (i, j)),
            pl.BlockSpec((bm, 128), lambda i, j: (i, 0)),
            pl.BlockSpec((bm, n), lambda i, j: (i, 0)),
        ],
        out_shape=[
            jax.ShapeDtypeStruct((m, k_pad), _F8),
            jax.ShapeDtypeStruct((m, 128), jnp.float32),
            jax.ShapeDtypeStruct((m, n), out_dtype),
        ],
        scratch_shapes=[
            pltpu.VMEM((bm, n), jnp.float32),
            pltpu.VMEM((bm, 128), jnp.float32),
        ],
        compiler_params=pltpu.CompilerParams(
            dimension_semantics=("parallel", "arbitrary"),
        ),
    )(adj, s_q, scale_row, corr_row, bias2d)


def _adj_f8_body(a_ref, s_ref, rs_ref, sv_ref, cr_ref, b_ref, o_ref,
                 acc_ref, *, nk, bk, relu):
    j = pl.program_id(1)

    @pl.when(j == 0)
    def _init():
        acc_ref[...] = jnp.zeros_like(acc_ref)

    s = s_ref[pl.ds(j * bk, bk), :]
    acc_ref[...] += jax.lax.dot(
        a_ref[...], s, preferred_element_type=jnp.float32)

    @pl.when(j == nk - 1)
    def _done():
        r = (acc_ref[...] * sv_ref[...]
             + rs_ref[:, 0:1] * cr_ref[...] + b_ref[...])
        if relu:
            r = jnp.maximum(r, 0.0)
        o_ref[...] = r.astype(o_ref.dtype)


def _adj_matmul_f8(adj_q, row_sums, s_q, scale_row, corr_row, bias2d, relu,
                   out_dtype, bm, bk):
    m, k_pad = adj_q.shape
    n = s_q.shape[1]
    nk = k_pad // bk
    grid = (pl.cdiv(m, bm), nk)
    return pl.pallas_call(
        functools.partial(_adj_f8_body, nk=nk, bk=bk, relu=relu),
        grid=grid,
        in_specs=[
            pl.BlockSpec((bm, bk), lambda i, j: (i, j)),
            pl.BlockSpec((k_pad, n), lambda i, j: (0, 0)),
            pl.BlockSpec((bm, 128), lambda i, j: (i, 0)),
            pl.BlockSpec((1, n), lambda i, j: (0, 0)),
            pl.BlockSpec((1, n), lambda i, j: (0, 0)),
            pl.BlockSpec((1, n), lambda i, j: (0, 0)),
        ],
        out_specs=pl.BlockSpec((bm, n), lambda i, j: (i, 0)),
        out_shape=jax.ShapeDtypeStruct((m, n), out_dtype),
        scratch_shapes=[pltpu.VMEM((bm, n), jnp.float32)],
        compiler_params=pltpu.CompilerParams(
            dimension_semantics=("parallel", "arbitrary"),
        ),
    )(adj_q, s_q, row_sums, scale_row, corr_row, bias2d)


def _gcn_forward(x, adj, layer_params, bm, bk):
    n_nodes = adj.shape[0]
    k_pad = pl.cdiv(n_nodes, bk) * bk
    n_layers = len(layer_params)
    h = x
    adj_q = None
    row_sums = None
    for i, (w, b) in enumerate(layer_params):
        relu = i < n_layers - 1
        n = w.shape[1]
        s_bf, mx, cs = _support_matmul(h, w, k_pad, bm)
        smax = jnp.maximum(jnp.max(mx), jnp.float32(1e-30))
        inv_row = jnp.full((1, n), _F8_TARGET, jnp.float32) / smax
        sigma = smax / _F8_TARGET
        s_q, cs_q = _quant_support(s_bf, inv_row, bm)
        colsum_s = jnp.sum(cs, axis=0)
        colsum_q = jnp.sum(cs_q, axis=0)
        corr_row = (colsum_s - sigma * colsum_q) / jnp.float32(k_pad)
        scale_row = jnp.full((1, n), 1.0, jnp.float32) * sigma
        out_dtype = jnp.bfloat16 if relu else jnp.float32
        b2 = b.reshape(1, -1)
        if i == 0:
            adj_q, row_sums, h = _adj_matmul_quantize(
                adj, s_q, scale_row, corr_row, b2, relu, out_dtype, bm, bk)
        else:
            h = _adj_matmul_f8(adj_q, row_sums, s_q, scale_row, corr_row,
                               b2, relu, out_dtype, bm, bk)
    return h


def kernel(x, adj, W1, b1, W2, b2, W3, b3, W4, b4):
    return _gcn_forward(
        x, adj, [(W1, b1), (W2, b2), (W3, b3), (W4, b4)], bm=_BM, bk=_BK
    )


# speedup vs baseline: 2.0397x; 1.2338x over previous
import functools

import jax
import jax.numpy as jnp
from jax.experimental import pallas as pl
from jax.experimental.pallas import tpu as pltpu

_BM = 1024
_BK = 2048
_F8_TARGET = 240.0
_F8 = jnp.float8_e4m3fn


def _support_body(h_ref, w_ref, o_ref, mx_ref, cs_ref, *, m_valid, bm):
    i = pl.program_id(0)
    h = h_ref[...].astype(jnp.bfloat16)
    w = w_ref[...].astype(jnp.bfloat16)
    o = jax.lax.dot(h, w, preferred_element_type=jnp.float32)
    row = jax.lax.broadcasted_iota(jnp.int32, o.shape, 0) + i * bm
    o = jnp.where(row < m_valid, o, 0.0)
    o_ref[...] = o.astype(o_ref.dtype)
    mx_ref[...] = jnp.full(mx_ref.shape, jnp.max(jnp.abs(o)), jnp.float32)
    cs_ref[...] = jnp.sum(o, axis=0, keepdims=True)[None]


def _support_matmul(h, w, m_pad, bm):
    m, k = h.shape
    n = w.shape[1]
    ng = m_pad // bm
    return pl.pallas_call(
        functools.partial(_support_body, m_valid=m, bm=bm),
        grid=(ng,),
        in_specs=[
            pl.BlockSpec((bm, k), lambda i: (i, 0)),
            pl.BlockSpec((k, n), lambda i: (0, 0)),
        ],
        out_specs=[
            pl.BlockSpec((bm, n), lambda i: (i, 0)),
            pl.BlockSpec((1, 1, 128), lambda i: (i, 0, 0)),
            pl.BlockSpec((1, 1, n), lambda i: (i, 0, 0)),
        ],
        out_shape=[
            jax.ShapeDtypeStruct((m_pad, n), jnp.bfloat16),
            jax.ShapeDtypeStruct((ng, 1, 128), jnp.float32),
            jax.ShapeDtypeStruct((ng, 1, n), jnp.float32),
        ],
        compiler_params=pltpu.CompilerParams(
            dimension_semantics=("parallel",),
        ),
    )(h, w)


def _quant_body(s_ref, inv_ref, q_ref, cs_ref):
    q = (s_ref[...].astype(jnp.float32) * inv_ref[...]).astype(_F8)
    q_ref[...] = q
    cs_ref[...] = jnp.sum(q.astype(jnp.float32), axis=0, keepdims=True)[None]


def _quant_support(s, inv_row, bm):
    m_pad, n = s.shape
    ng = m_pad // bm
    return pl.pallas_call(
        _quant_body,
        grid=(ng,),
        in_specs=[
            pl.BlockSpec((bm, n), lambda i: (i, 0)),
            pl.BlockSpec((1, n), lambda i: (0, 0)),
        ],
        out_specs=[
            pl.BlockSpec((bm, n), lambda i: (i, 0)),
            pl.BlockSpec((1, 1, n), lambda i: (i, 0, 0)),
        ],
        out_shape=[
            jax.ShapeDtypeStruct((m_pad, n), _F8),
            jax.ShapeDtypeStruct((ng, 1, n), jnp.float32),
        ],
        compiler_params=pltpu.CompilerParams(
            dimension_semantics=("parallel",),
        ),
    )(s, inv_row)


def _adj_q_body(a_ref, s_ref, sv_ref, cr_ref, b_ref, w_ref,
                q_ref, rs_ref, sn_ref, mx_ref, cs_ref,
                acc_ref, rsacc_ref, *, nk, bk, k_valid, bm):
    i = pl.program_id(0)
    j = pl.program_id(1)
    a = a_ref[...]
    col = jax.lax.broadcasted_iota(jnp.int32, a.shape, 1) + j * bk
    a = jnp.where(col < k_valid, a, 0.0)
    qa = a.astype(_F8)
    q_ref[...] = qa

    @pl.when(j == 0)
    def _init():
        acc_ref[...] = jnp.zeros_like(acc_ref)
        rsacc_ref[...] = jnp.zeros_like(rsacc_ref)

    rsacc_ref[...] += jnp.broadcast_to(
        jnp.sum(qa.astype(jnp.float32), axis=1, keepdims=True),
        rsacc_ref.shape)
    s = s_ref[pl.ds(j * bk, bk), :]
    acc_ref[...] += jax.lax.dot(qa, s, preferred_element_type=jnp.float32)

    @pl.when(j == nk - 1)
    def _done():
        rs_ref[...] = rsacc_ref[...]
        r = (acc_ref[...] * sv_ref[...]
             + rsacc_ref[:, 0:1] * cr_ref[...] + b_ref[...])
        r = jnp.maximum(r, 0.0)
        row = jax.lax.broadcasted_iota(jnp.int32, r.shape, 0) + i * bm
        h = jnp.where(row < k_valid, r, 0.0).astype(jnp.bfloat16)
        sn = jax.lax.dot(h, w_ref[...].astype(jnp.bfloat16),
                         preferred_element_type=jnp.float32)
        sn_ref[...] = sn.astype(jnp.bfloat16)
        mx_ref[...] = jnp.full(mx_ref.shape, jnp.max(jnp.abs(sn)), jnp.float32)
        cs_ref[...] = jnp.sum(sn, axis=0, keepdims=True)[None]


def _adj_matmul_quantize(adj, s_q, scale_row, corr_row, bias2d, w_next,
                         bm, bk):
    m, kdim = adj.shape
    k_pad, n = s_q.shape
    n2 = w_next.shape[1]
    nk = k_pad // bk
    ng = k_pad // bm
    grid = (ng, nk)
    return pl.pallas_call(
        functools.partial(_adj_q_body, nk=nk, bk=bk, k_valid=kdim, bm=bm),
        grid=grid,
        in_specs=[
            pl.BlockSpec((bm, bk), lambda i, j: (i, j)),
            pl.BlockSpec((k_pad, n), lambda i, j: (0, 0)),
            pl.BlockSpec((1, n), lambda i, j: (0, 0)),
            pl.BlockSpec((1, n), lambda i, j: (0, 0)),
            pl.BlockSpec((1, n), lambda i, j: (0, 0)),
            pl.BlockSpec((n, n2), lambda i, j: (0, 0)),
        ],
        out_specs=[
            pl.BlockSpec((bm, bk), lambda i, j: (i, j)),
            pl.BlockSpec((bm, 128), lambda i, j: (i, 0)),
            pl.BlockSpec((bm, n2), lambda i, j: (i, 0)),
            pl.BlockSpec((1, 1, 128), lambda i, j: (i, 0, 0)),
            pl.BlockSpec((1, 1, n2), lambda i, j: (i, 0, 0)),
        ],
        out_shape=[
            jax.ShapeDtypeStruct((m, k_pad), _F8),
            jax.ShapeDtypeStruct((m, 128), jnp.float32),
            jax.ShapeDtypeStruct((k_pad, n2), jnp.bfloat16),
            jax.ShapeDtypeStruct((ng, 1, 128), jnp.float32),
            jax.ShapeDtypeStruct((ng, 1, n2), jnp.float32),
        ],
        scratch_shapes=[
            pltpu.VMEM((bm, n), jnp.float32),
            pltpu.VMEM((bm, 128), jnp.float32),
        ],
        compiler_params=pltpu.CompilerParams(
            dimension_semantics=("parallel", "arbitrary"),
        ),
    )(adj, s_q, scale_row, corr_row, bias2d, w_next)


def _adj_f8_mid_body(a_ref, s_ref, rs_ref, sv_ref, cr_ref, b_ref, w_ref,
                     sn_ref, mx_ref, cs_ref, *, m_valid, bm):
    i = pl.program_id(0)
    acc = jax.lax.dot(a_ref[...], s_ref[...],
                      preferred_element_type=jnp.float32)
    r = acc * sv_ref[...] + rs_ref[:, 0:1] * cr_ref[...] + b_ref[...]
    r = jnp.maximum(r, 0.0)
    row = jax.lax.broadcasted_iota(jnp.int32, r.shape, 0) + i * bm
    h = jnp.where(row < m_valid, r, 0.0).astype(jnp.bfloat16)
    sn = jax.lax.dot(h, w_ref[...].astype(jnp.bfloat16),
                     preferred_element_type=jnp.float32)
    sn_ref[...] = sn.astype(jnp.bfloat16)
    mx_ref[...] = jnp.full(mx_ref.shape, jnp.max(jnp.abs(sn)), jnp.float32)
    cs_ref[...] = jnp.sum(sn, axis=0, keepdims=True)[None]


def _adj_matmul_f8_mid(adj_q, row_sums, s_q, scale_row, corr_row, bias2d,
                       w_next, m_valid, bm):
    m, k_pad = adj_q.shape
    n = s_q.shape[1]
    n2 = w_next.shape[1]
    ng = k_pad // bm
    return pl.pallas_call(
        functools.partial(_adj_f8_mid_body, m_valid=m_valid, bm=bm),
        grid=(ng,),
        in_specs=[
            pl.BlockSpec((bm, k_pad), lambda i: (i, 0)),
            pl.BlockSpec((k_pad, n), lambda i: (0, 0)),
            pl.BlockSpec((bm, 128), lambda i: (i, 0)),
            pl.BlockSpec((1, n), lambda i: (0, 0)),
            pl.BlockSpec((1, n), lambda i: (0, 0)),
            pl.BlockSpec((1, n), lambda i: (0, 0)),
            pl.BlockSpec((n, n2), lambda i: (0, 0)),
        ],
        out_specs=[
            pl.BlockSpec((bm, n2), lambda i: (i, 0)),
            pl.BlockSpec((1, 1, 128), lambda i: (i, 0, 0)),
            pl.BlockSpec((1, 1, n2), lambda i: (i, 0, 0)),
        ],
        out_shape=[
            jax.ShapeDtypeStruct((k_pad, n2), jnp.bfloat16),
            jax.ShapeDtypeStruct((ng, 1, 128), jnp.float32),
            jax.ShapeDtypeStruct((ng, 1, n2), jnp.float32),
        ],
        compiler_params=pltpu.CompilerParams(
            dimension_semantics=("parallel",),
        ),
    )(adj_q, s_q, row_sums, scale_row, corr_row, bias2d, w_next)


def _adj_f8_last_body(a_ref, s_ref, rs_ref, sv_ref, cr_ref, b_ref, o_ref):
    acc = jax.lax.dot(a_ref[...], s_ref[...],
                      preferred_element_type=jnp.float32)
    o_ref[...] = acc * sv_ref[...] + rs_ref[:, 0:1] * cr_ref[...] + b_ref[...]


def _adj_matmul_f8_last(adj_q, row_sums, s_q, scale_row, corr_row, bias2d,
                        bm):
    m, k_pad = adj_q.shape
    n = s_q.shape[1]
    ng = k_pad // bm
    return pl.pallas_call(
        _adj_f8_last_body,
        grid=(ng,),
        in_specs=[
            pl.BlockSpec((bm, k_pad), lambda i: (i, 0)),
            pl.BlockSpec((k_pad, n), lambda i: (0, 0)),
            pl.BlockSpec((bm, 128), lambda i: (i, 0)),
            pl.BlockSpec((1, n), lambda i: (0, 0)),
            pl.BlockSpec((1, n), lambda i: (0, 0)),
            pl.BlockSpec((1, n), lambda i: (0, 0)),
        ],
        out_specs=pl.BlockSpec((bm, n), lambda i: (i, 0)),
        out_shape=jax.ShapeDtypeStruct((m, n), jnp.float32),
        compiler_params=pltpu.CompilerParams(
            dimension_semantics=("parallel",),
        ),
    )(adj_q, s_q, row_sums, scale_row, corr_row, bias2d)


def _layer_scales(mx, cs, cs_q, n, k_pad, smax):
    sigma = smax / _F8_TARGET
    colsum_s = jnp.sum(cs, axis=0)
    colsum_q = jnp.sum(cs_q, axis=0)
    corr_row = (colsum_s - sigma * colsum_q) / jnp.float32(k_pad)
    scale_row = jnp.full((1, n), 1.0, jnp.float32) * sigma
    return scale_row, corr_row


def _gcn_forward(x, adj, layer_params, bm, bk):
    n_nodes = adj.shape[0]
    k_pad = pl.cdiv(n_nodes, bk) * bk

    (w1, b1), (w2, b2), (w3, b3), (w4, b4) = layer_params

    s_bf, mx, cs = _support_matmul(x, w1, k_pad, bm)
    smax = jnp.maximum(jnp.max(mx), jnp.float32(1e-30))
    n1 = w1.shape[1]
    inv_row = jnp.full((1, n1), _F8_TARGET, jnp.float32) / smax
    s_q, cs_q = _quant_support(s_bf, inv_row, bm)
    sv, cr = _layer_scales(mx, cs, cs_q, n1, k_pad, smax)
    adj_q, row_sums, s_bf, mx, cs = _adj_matmul_quantize(
        adj, s_q, sv, cr, b1.reshape(1, -1), w2.astype(jnp.bfloat16), bm, bk)

    for w, b, w_next in ((w2, b2, w3), (w3, b3, w4)):
        n = w.shape[1]
        smax = jnp.maximum(jnp.max(mx), jnp.float32(1e-30))
        inv_row = jnp.full((1, n), _F8_TARGET, jnp.float32) / smax
        s_q, cs_q = _quant_support(s_bf, inv_row, bm)
        sv, cr = _layer_scales(mx, cs, cs_q, n, k_pad, smax)
        s_bf, mx, cs = _adj_matmul_f8_mid(
            adj_q, row_sums, s_q, sv, cr, b.reshape(1, -1),
            w_next.astype(jnp.bfloat16), n_nodes, bm)

    n = w4.shape[1]
    smax = jnp.maximum(jnp.max(mx), jnp.float32(1e-30))
    inv_row = jnp.full((1, n), _F8_TARGET, jnp.float32) / smax
    s_q, cs_q = _quant_support(s_bf, inv_row, bm)
    sv, cr = _layer_scales(mx, cs, cs_q, n, k_pad, smax)
    return _adj_matmul_f8_last(
        adj_q, row_sums, s_q, sv, cr, b4.reshape(1, -1), bm)


def kernel(x, adj, W1, b1, W2, b2, W3, b3, W4, b4):
    return _gcn_forward(
        x, adj, [(W1, b1), (W2, b2), (W3, b3), (W4, b4)], bm=_BM, bk=_BK
    )
